# Initial kernel scaffold; baseline (speedup 1.0000x reference)
import jax
import jax.numpy as jnp
from jax.experimental import pallas as pl


def kernel(entity_reps, restriction_maps, edge_index, entity_types):
    raise NotImplementedError("write your pallas kernel here")


# speedup vs baseline: 281.5287x; 281.5287x over previous
import functools

import jax
import jax.numpy as jnp
from jax import lax
from jax.experimental import pallas as pl
from jax.experimental.pallas import tpu as pltpu
from jax.experimental.pallas import tpu_sc as plsc

N_NODES = 50000
N_EDGES = 1600000
N_TYPES = 10

NC = 2
NS = 16
L = 16
NW = NC * NS

NPAD = 51200
SW = NPAD // NS
EPW = N_EDGES // NW
C = 2000
NCHUNK = EPW // C

_mesh = plsc.VectorSubcoreMesh(core_axis_name="c", subcore_axis_name="s")


def _deg_body(t_hbm, deg_hbm, idxbuf, hist, accbuf, tmpbuf, slab, sem):
    c = lax.axis_index("c")
    s = lax.axis_index("s")
    w = c * NS + s
    base = w * EPW
    zeros = jnp.zeros((L,), jnp.float32)
    ones = jnp.ones((L,), jnp.float32)

    @pl.loop(0, NPAD // L)
    def _zero(i):
        hist[pl.ds(i * L, L)] = zeros

    @pl.loop(0, NCHUNK)
    def _chunk(j):
        pltpu.async_copy(t_hbm.at[pl.ds(base + j * C, C)], idxbuf, sem).wait()

        @pl.loop(0, C // L)
        def _step(i):
            tv = idxbuf[pl.ds(i * L, L)]
            plsc.addupdate_scatter(hist, [tv], ones)

    pltpu.sync_copy(hist, slab.at[s])
    plsc.subcore_barrier()

    @pl.loop(0, SW // L)
    def _zacc(i):
        accbuf[pl.ds(i * L, L)] = zeros

    @pl.loop(0, NS)
    def _red(k):
        pltpu.async_copy(slab.at[k, pl.ds(s * SW, SW)], tmpbuf, sem).wait()

        @pl.loop(0, SW // L)
        def _add(i):
            accbuf[pl.ds(i * L, L)] = (
                accbuf[pl.ds(i * L, L)] + tmpbuf[pl.ds(i * L, L)]
            )

    pltpu.sync_copy(accbuf, deg_hbm.at[c, pl.ds(s * SW, SW)])


_deg_call = pl.kernel(
    _deg_body,
    out_type=jax.ShapeDtypeStruct((NC, NPAD), jnp.float32),
    mesh=_mesh,
    scratch_types=[
        pltpu.VMEM((C,), jnp.int32),
        pltpu.VMEM((NPAD,), jnp.float32),
        pltpu.VMEM((SW,), jnp.float32),
        pltpu.VMEM((SW,), jnp.float32),
        pltpu.VMEM_SHARED((NS, NPAD), jnp.float32),
        pltpu.SemaphoreType.DMA,
    ],
    compiler_params=pltpu.CompilerParams(needs_layout_passes=False),
)


def _energy_body(p0_hbm, p1_hbm, tab_hbm, h_hbm, t_hbm, out_hbm,
                 p0v, p1v, tabv, hbuf0, tbuf0, hbuf1, tbuf1, accv,
                 semt, sem0, sem1):
    c = lax.axis_index("c")
    s = lax.axis_index("s")
    w = c * NS + s
    base = w * EPW

    pltpu.async_copy(p0_hbm, p0v, semt)
    pltpu.async_copy(p1_hbm, p1v, semt)
    pltpu.async_copy(tab_hbm, tabv, semt)

    def fire(cid, hb, tb, sem):
        pltpu.async_copy(h_hbm.at[pl.ds(base + cid * C, C)], hb, sem)
        pltpu.async_copy(t_hbm.at[pl.ds(base + cid * C, C)], tb, sem)

    def drain(cid, hb, tb, sem):
        pltpu.make_async_copy(h_hbm.at[pl.ds(base + cid * C, C)], hb, sem).wait()
        pltpu.make_async_copy(t_hbm.at[pl.ds(base + cid * C, C)], tb, sem).wait()

    fire(0, hbuf0, tbuf0, sem0)
    pltpu.make_async_copy(p0_hbm, p0v, semt).wait()
    pltpu.make_async_copy(p1_hbm, p1v, semt).wait()
    pltpu.make_async_copy(tab_hbm, tabv, semt).wait()

    def compute(hb, tb, acc_in):
        @pl.loop(0, C // L, init_carry=acc_in, unroll=2)
        def _step(i, acc):
            hv = hb[pl.ds(i * L, L)]
            tv = tb[pl.ds(i * L, L)]
            u0 = plsc.load_gather(p0v, [hv])
            u1 = plsc.load_gather(p1v, [hv])
            v0 = plsc.load_gather(p0v, [tv])
            v1 = plsc.load_gather(p1v, [tv])
            th = plsc.bitcast(u0, jnp.int32) & 15
            tt = plsc.bitcast(v0, jnp.int32) & 15
            pa = th * N_TYPES + tt
            pb = tt * N_TYPES + th
            a00 = plsc.load_gather(tabv, [pa])
            a01 = plsc.load_gather(tabv, [pa + 100])
            a10 = plsc.load_gather(tabv, [pa + 200])
            a11 = plsc.load_gather(tabv, [pa + 300])
            b00 = plsc.load_gather(tabv, [pb])
            b01 = plsc.load_gather(tabv, [pb + 100])
            b10 = plsc.load_gather(tabv, [pb + 200])
            b11 = plsc.load_gather(tabv, [pb + 300])
            w0 = a00 * u0 + a10 * u1 - (b00 * v0 + b10 * v1)
            w1 = a01 * u0 + a11 * u1 - (b01 * v0 + b11 * v1)
            return acc + w0 * w0 + w1 * w1
        return _step

    acc = jnp.zeros((L,), jnp.float32)

    @pl.loop(0, (NCHUNK - 1) // 2, init_carry=acc)
    def _outer(k, acc):
        j = 2 * k
        fire(j + 1, hbuf1, tbuf1, sem1)
        drain(j, hbuf0, tbuf0, sem0)
        acc = compute(hbuf0, tbuf0, acc)
        fire(j + 2, hbuf0, tbuf0, sem0)
        drain(j + 1, hbuf1, tbuf1, sem1)
        acc = compute(hbuf1, tbuf1, acc)
        return acc

    acc = _outer
    drain(NCHUNK - 1, hbuf0, tbuf0, sem0)
    acc = compute(hbuf0, tbuf0, acc)
    accv[...] = acc
    pltpu.sync_copy(accv, out_hbm.at[w])


_energy_call = pl.kernel(
    _energy_body,
    out_type=jax.ShapeDtypeStruct((NW, L), jnp.float32),
    mesh=_mesh,
    scratch_types=[
        pltpu.VMEM((N_NODES,), jnp.float32),
        pltpu.VMEM((N_NODES,), jnp.float32),
        pltpu.VMEM((400,), jnp.float32),
        pltpu.VMEM((C,), jnp.int32),
        pltpu.VMEM((C,), jnp.int32),
        pltpu.VMEM((C,), jnp.int32),
        pltpu.VMEM((C,), jnp.int32),
        pltpu.VMEM((L,), jnp.float32),
        pltpu.SemaphoreType.DMA,
        pltpu.SemaphoreType.DMA,
        pltpu.SemaphoreType.DMA,
    ],
    compiler_params=pltpu.CompilerParams(needs_layout_passes=False),
)


def kernel(entity_reps, restriction_maps, edge_index, entity_types):
    h = edge_index[0]
    t = edge_index[1]

    deg2 = _deg_call(t)
    deg = deg2[0, :N_NODES] + deg2[1, :N_NODES]
    invd = jnp.where(deg > 0, lax.rsqrt(deg), 0.0)
    p0 = entity_reps[0] * invd
    p1 = entity_reps[1] * invd
    p0s = lax.bitcast_convert_type(
        (lax.bitcast_convert_type(p0, jnp.int32) & ~15) | entity_types,
        jnp.float32)

    norms = jnp.sqrt(jnp.sum(restriction_maps ** 2, axis=(-2, -1)))
    nm = restriction_maps / norms[..., None, None]
    tab = jnp.concatenate([
        nm[..., 0, 0].ravel(), nm[..., 0, 1].ravel(),
        nm[..., 1, 0].ravel(), nm[..., 1, 1].ravel(),
    ])

    parts = _energy_call(p0s, p1, tab, h, t)
    return jnp.sum(parts)


# speedup vs baseline: 422.5024x; 1.5007x over previous
import jax
import jax.numpy as jnp
from jax import lax
from jax.experimental import pallas as pl
from jax.experimental.pallas import tpu as pltpu
from jax.experimental.pallas import tpu_sc as plsc

N_NODES = 50000
N_EDGES = 1600000
N_TYPES = 10

NC = 2
NS = 16
L = 16
NW = NC * NS

NPAD = 51200
SW = NPAD // NS
SUBW = 800
C = 2000

EPW = N_EDGES // NW
NCH = EPW // C

_mesh = plsc.VectorSubcoreMesh(core_axis_name="c", subcore_axis_name="s")
_params = pltpu.CompilerParams(needs_layout_passes=False)

_F32 = jnp.float32
_I32 = jnp.int32


def _rsqrt_newton(d):
    i = jnp.int32(0x5F3759DF) - (plsc.bitcast(d, _I32) >> 1)
    y = plsc.bitcast(i, _F32)
    half_d = d * 0.5
    for _ in range(3):
        y = y * (1.5 - half_d * y * y)
    return y


def _pack2bf16(a, b):
    ia = plsc.bitcast(a, _I32) + jnp.int32(0x8000)
    ib = plsc.bitcast(b, _I32) + jnp.int32(0x8000)
    return ((ia >> 16) & jnp.int32(0xFFFF)) | (ib & jnp.int32(-65536))


def _k1_body(ei_hbm, deg_hbm, idx0, idx1, hist, accb, tmp0, tmp1,
             slab, sem0, sem1):
    c = lax.axis_index("c")
    s = lax.axis_index("s")
    w = c * NS + s
    base = w * EPW
    zeros = jnp.zeros((L,), _F32)
    ones = jnp.ones((L,), _F32)

    @pl.loop(0, NPAD // L)
    def _zero(i):
        hist[pl.ds(i * L, L)] = zeros

    def fire(j, buf, sm):
        pltpu.async_copy(ei_hbm.at[pl.ds(N_EDGES + base + j * C, C)], buf, sm)

    def drain(j, buf, sm):
        pltpu.make_async_copy(
            ei_hbm.at[pl.ds(N_EDGES + base + j * C, C)], buf, sm).wait()

    def scatter(buf):
        @pl.loop(0, C // L)
        def _st(i):
            tv = buf[pl.ds(i * L, L)]
            plsc.addupdate_scatter(hist, [tv], ones)

    fire(0, idx0, sem0)

    @pl.loop(0, (NCH - 1) // 2)
    def _outer(k):
        j = 2 * k
        fire(j + 1, idx1, sem1)
        drain(j, idx0, sem0)
        scatter(idx0)
        fire(j + 2, idx0, sem0)
        drain(j + 1, idx1, sem1)
        scatter(idx1)

    drain(NCH - 1, idx0, sem0)
    scatter(idx0)

    pltpu.sync_copy(hist, slab.at[s])
    plsc.subcore_barrier()

    @pl.loop(0, SW // L)
    def _zacc(i):
        accb[pl.ds(i * L, L)] = zeros

    def rfire(k, buf, sm):
        pltpu.async_copy(slab.at[k, pl.ds(s * SW, SW)], buf, sm)

    def rdrain(k, buf, sm):
        pltpu.make_async_copy(slab.at[k, pl.ds(s * SW, SW)], buf, sm).wait()

    def radd(buf):
        @pl.loop(0, SW // L)
        def _a(i):
            accb[pl.ds(i * L, L)] = accb[pl.ds(i * L, L)] + buf[pl.ds(i * L, L)]

    rfire(0, tmp0, sem0)

    @pl.loop(0, NS // 2)
    def _red(k):
        j = 2 * k
        rfire(j + 1, tmp1, sem1)
        rdrain(j, tmp0, sem0)
        radd(tmp0)

        @pl.when(k < NS // 2 - 1)
        def _():
            rfire(j + 2, tmp0, sem0)

        rdrain(j + 1, tmp1, sem1)
        radd(tmp1)

    pltpu.sync_copy(accb, deg_hbm.at[pl.ds(c * NPAD + s * SW, SW)])


_k1 = pl.kernel(
    _k1_body,
    out_type=jax.ShapeDtypeStruct((NC * NPAD,), _F32),
    mesh=_mesh,
    scratch_types=[
        pltpu.VMEM((C,), _I32),
        pltpu.VMEM((C,), _I32),
        pltpu.VMEM((NPAD,), _F32),
        pltpu.VMEM((SW,), _F32),
        pltpu.VMEM((SW,), _F32),
        pltpu.VMEM((SW,), _F32),
        pltpu.VMEM_SHARED((NS, NPAD), _F32),
        pltpu.SemaphoreType.DMA,
        pltpu.SemaphoreType.DMA,
    ],
    compiler_params=_params,
)


def _k2_body(ei_hbm, deg_hbm, x_hbm, ty_hbm, rm_hbm, out_hbm,
             p0v, p1v, rmv, tabv, hb0, tb0, hb1, tb1, accv,
             d0b, d1b, xb0, xb1, tyb, pslab,
             semt, sem0, sem1):
    c = lax.axis_index("c")
    s = lax.axis_index("s")
    w = c * NS + s
    base = w * EPW

    @pl.loop(0, SW // SUBW)
    def _prep(r):
        o = s * SW + r * SUBW
        pltpu.async_copy(deg_hbm.at[pl.ds(o, SUBW)], d0b, semt)
        pltpu.async_copy(deg_hbm.at[pl.ds(NPAD + o, SUBW)], d1b, semt)
        pltpu.async_copy(x_hbm.at[pl.ds(o, SUBW)], xb0, semt)
        pltpu.async_copy(x_hbm.at[pl.ds(NPAD + o, SUBW)], xb1, semt)
        pltpu.async_copy(ty_hbm.at[pl.ds(o, SUBW)], tyb, semt)
        pltpu.make_async_copy(deg_hbm.at[pl.ds(o, SUBW)], d0b, semt).wait()
        pltpu.make_async_copy(deg_hbm.at[pl.ds(NPAD + o, SUBW)], d1b, semt).wait()
        pltpu.make_async_copy(x_hbm.at[pl.ds(o, SUBW)], xb0, semt).wait()
        pltpu.make_async_copy(x_hbm.at[pl.ds(NPAD + o, SUBW)], xb1, semt).wait()
        pltpu.make_async_copy(ty_hbm.at[pl.ds(o, SUBW)], tyb, semt).wait()

        @pl.loop(0, SUBW // L)
        def _p(i):
            sl = pl.ds(i * L, L)
            d = d0b[sl] + d1b[sl]
            y = _rsqrt_newton(d)
            invd = jnp.where(d > 0.5, y, 0.0)
            p0 = xb0[sl] * invd
            p1 = xb1[sl] * invd
            xb0[sl] = plsc.bitcast(
                (plsc.bitcast(p0, _I32) & jnp.int32(~15)) | tyb[sl], _F32)
            xb1[sl] = p1

        pltpu.sync_copy(xb0, pslab.at[pl.ds(o, SUBW)])
        pltpu.sync_copy(xb1, pslab.at[pl.ds(NPAD + o, SUBW)])

    pltpu.sync_copy(rm_hbm, rmv)
    for st in range(7):
        pidx = lax.iota(_I32, 16) + jnp.int32(st * L)
        pc = jnp.minimum(pidx, jnp.int32(N_TYPES * N_TYPES - 1))
        q = pc * 4
        m00 = plsc.load_gather(rmv, [q])
        m01 = plsc.load_gather(rmv, [q + 1])
        m10 = plsc.load_gather(rmv, [q + 2])
        m11 = plsc.load_gather(rmv, [q + 3])
        y = _rsqrt_newton(m00 * m00 + m01 * m01 + m10 * m10 + m11 * m11)
        w0 = _pack2bf16(m00 * y, m10 * y)
        w1 = _pack2bf16(m01 * y, m11 * y)
        plsc.store_scatter(tabv, [2 * pc], w0)
        plsc.store_scatter(tabv, [2 * pc + 1], w1)

    plsc.subcore_barrier()
    pltpu.async_copy(pslab.at[pl.ds(0, NPAD)], p0v, semt)
    pltpu.async_copy(pslab.at[pl.ds(NPAD, NPAD)], p1v, semt)

    def fire(j, hb, tb, sm):
        pltpu.async_copy(ei_hbm.at[pl.ds(base + j * C, C)], hb, sm)
        pltpu.async_copy(ei_hbm.at[pl.ds(N_EDGES + base + j * C, C)], tb, sm)

    def drain(j, hb, tb, sm):
        pltpu.make_async_copy(ei_hbm.at[pl.ds(base + j * C, C)], hb, sm).wait()
        pltpu.make_async_copy(
            ei_hbm.at[pl.ds(N_EDGES + base + j * C, C)], tb, sm).wait()

    fire(0, hb0, tb0, sem0)
    pltpu.make_async_copy(pslab.at[pl.ds(0, NPAD)], p0v, semt).wait()
    pltpu.make_async_copy(pslab.at[pl.ds(NPAD, NPAD)], p1v, semt).wait()

    mhi = jnp.int32(-65536)

    def compute(hb, tb, acc_in):
        @pl.loop(0, C // L, init_carry=acc_in, unroll=2)
        def _step(i, acc):
            sl = pl.ds(i * L, L)
            hv = hb[sl]
            tv = tb[sl]
            u0 = plsc.load_gather(p0v, [hv])
            u1 = plsc.load_gather(p1v, [hv])
            v0 = plsc.load_gather(p0v, [tv])
            v1 = plsc.load_gather(p1v, [tv])
            th = plsc.bitcast(u0, _I32) & 15
            tt = plsc.bitcast(v0, _I32) & 15
            pa = (th * N_TYPES + tt) * 2
            pb = (tt * N_TYPES + th) * 2
            wa0 = plsc.load_gather(tabv, [pa])
            wa1 = plsc.load_gather(tabv, [pa + 1])
            wb0 = plsc.load_gather(tabv, [pb])
            wb1 = plsc.load_gather(tabv, [pb + 1])
            a00 = plsc.bitcast(wa0 << 16, _F32)
            a10 = plsc.bitcast(wa0 & mhi, _F32)
            a01 = plsc.bitcast(wa1 << 16, _F32)
            a11 = plsc.bitcast(wa1 & mhi, _F32)
            b00 = plsc.bitcast(wb0 << 16, _F32)
            b10 = plsc.bitcast(wb0 & mhi, _F32)
            b01 = plsc.bitcast(wb1 << 16, _F32)
            b11 = plsc.bitcast(wb1 & mhi, _F32)
            w0 = a00 * u0 + a10 * u1 - (b00 * v0 + b10 * v1)
            w1 = a01 * u0 + a11 * u1 - (b01 * v0 + b11 * v1)
            return acc + w0 * w0 + w1 * w1
        return _step

    acc = jnp.zeros((L,), _F32)

    @pl.loop(0, (NCH - 1) // 2, init_carry=acc)
    def _outer(k, acc):
        j = 2 * k
        fire(j + 1, hb1, tb1, sem1)
        drain(j, hb0, tb0, sem0)
        acc = compute(hb0, tb0, acc)
        fire(j + 2, hb0, tb0, sem0)
        drain(j + 1, hb1, tb1, sem1)
        acc = compute(hb1, tb1, acc)
        return acc

    acc = _outer
    drain(NCH - 1, hb0, tb0, sem0)
    acc = compute(hb0, tb0, acc)
    accv[...] = acc
    pltpu.sync_copy(accv, out_hbm.at[w])


_k2 = pl.kernel(
    _k2_body,
    out_type=jax.ShapeDtypeStruct((NW, L), _F32),
    mesh=_mesh,
    scratch_types=[
        pltpu.VMEM((NPAD,), _F32),
        pltpu.VMEM((NPAD,), _F32),
        pltpu.VMEM((400,), _F32),
        pltpu.VMEM((256,), _I32),
        pltpu.VMEM((C,), _I32),
        pltpu.VMEM((C,), _I32),
        pltpu.VMEM((C,), _I32),
        pltpu.VMEM((C,), _I32),
        pltpu.VMEM((L,), _F32),
        pltpu.VMEM((SUBW,), _F32),
        pltpu.VMEM((SUBW,), _F32),
        pltpu.VMEM((SUBW,), _F32),
        pltpu.VMEM((SUBW,), _F32),
        pltpu.VMEM((SUBW,), _I32),
        pltpu.VMEM_SHARED((2 * NPAD,), _F32),
        pltpu.SemaphoreType.DMA,
        pltpu.SemaphoreType.DMA,
        pltpu.SemaphoreType.DMA,
    ],
    compiler_params=_params,
)


def kernel(entity_reps, restriction_maps, edge_index, entity_types):
    xpad = jnp.pad(entity_reps, ((0, 0), (0, NPAD - N_NODES))).reshape(-1)
    typad = jnp.pad(entity_types, (0, NPAD - N_NODES))
    rmflat = restriction_maps.reshape(-1)
    eiflat = edge_index.reshape(-1)
    deg = _k1(eiflat)
    parts = _k2(eiflat, deg, xpad, typad, rmflat)
    return jnp.sum(parts)
